# baseline (speedup 1.0000x reference)
import functools

import jax
import jax.numpy as jnp
from jax import lax
from jax.experimental import pallas as pl
from jax.experimental.pallas import tpu as pltpu
from jax.experimental.pallas import tpu_sc as plsc

NC = 2
NS = 16
LANES = 128
HALF = 64
DB = 8
DPAD = 128
DEGW = 128

_vector_mesh = plsc.VectorSubcoreMesh(
    core_axis_name="core", subcore_axis_name="subcore")


def _round_up(a, b):
    return (a + b - 1) // b * b


def _sc_degree(dstm, ones_row, zeros_nd, *, cpt, np_rows):
    rpt = np_rows // NS

    @functools.partial(
        pl.kernel,
        out_type=jax.ShapeDtypeStruct((NC, np_rows, DEGW), jnp.float32),
        mesh=_vector_mesh,
        scratch_types=[
            pltpu.VMEM((cpt, LANES), jnp.int32),
            pltpu.VMEM((LANES, DEGW), jnp.float32),
            pltpu.VMEM_SHARED((np_rows, DEGW), jnp.float32),
        ],
    )
    def k(dstm_hbm, ones_hbm, zeros_hbm, out_hbm, di, ones_v, acc):
        c = lax.axis_index("core")
        s = lax.axis_index("subcore")
        wid = s * NC + c
        pltpu.sync_copy(dstm_hbm.at[pl.ds(wid * cpt, cpt)], di)
        pltpu.sync_copy(ones_hbm, ones_v)
        pltpu.sync_copy(zeros_hbm.at[pl.ds(s * rpt, rpt)],
                        acc.at[pl.ds(s * rpt, rpt)])
        plsc.subcore_barrier()

        @pl.loop(0, cpt)
        def _(j):
            pltpu.sync_copy(ones_v, acc.at[di.at[j]], add=True)

        plsc.subcore_barrier()
        pltpu.sync_copy(acc.at[pl.ds(s * rpt, rpt)],
                        out_hbm.at[c].at[pl.ds(s * rpt, rpt)])

    return k(dstm, ones_row, zeros_nd)


def _sc_propagate(table, srcm, dstm, zeros_npd, *, cpt, np_rows):
    rpt = np_rows // NS

    @functools.partial(
        pl.kernel,
        out_type=jax.ShapeDtypeStruct((NC, np_rows, DPAD), jnp.float32),
        mesh=_vector_mesh,
        scratch_types=[
            pltpu.VMEM((cpt + 1, LANES), jnp.int32),
            pltpu.VMEM((cpt, LANES), jnp.int32),
            pltpu.VMEM((HALF, DPAD), jnp.float32),
            pltpu.VMEM((HALF, DPAD), jnp.float32),
            pltpu.VMEM_SHARED((np_rows, DPAD), jnp.float32),
            pltpu.SemaphoreType.DMA,
            pltpu.SemaphoreType.DMA,
            pltpu.SemaphoreType.DMA,
            pltpu.SemaphoreType.DMA,
        ],
    )
    def k(table_hbm, srcm_hbm, dstm_hbm, zeros_hbm, out_hbm,
          si, di, r0, r1, acc, g0, g1, a0, a1):
        c = lax.axis_index("core")
        s = lax.axis_index("subcore")
        wid = s * NC + c
        pltpu.sync_copy(srcm_hbm.at[pl.ds(wid * cpt, cpt)],
                        si.at[pl.ds(0, cpt)])
        pltpu.sync_copy(srcm_hbm.at[pl.ds(wid * cpt, 1)],
                        si.at[pl.ds(cpt, 1)])
        pltpu.sync_copy(dstm_hbm.at[pl.ds(wid * cpt, cpt)], di)
        pltpu.sync_copy(zeros_hbm.at[pl.ds(s * rpt, rpt)],
                        acc.at[pl.ds(s * rpt, rpt)])
        plsc.subcore_barrier()

        pltpu.async_copy(table_hbm.at[si.at[0, pl.ds(0, HALF)]], r0, g0)
        pltpu.async_copy(table_hbm.at[si.at[0, pl.ds(HALF, HALF)]], r1, g1)

        @pl.loop(0, cpt)
        def _(j):
            pltpu.make_async_copy(table_hbm.at[si.at[j, pl.ds(0, HALF)]],
                                  r0, g0).wait()
            pltpu.async_copy(r0, acc.at[di.at[j, pl.ds(0, HALF)]], a0,
                             add=True)
            pltpu.make_async_copy(table_hbm.at[si.at[j, pl.ds(HALF, HALF)]],
                                  r1, g1).wait()
            pltpu.async_copy(r1, acc.at[di.at[j, pl.ds(HALF, HALF)]], a1,
                             add=True)

            pltpu.make_async_copy(r0, acc.at[di.at[j, pl.ds(0, HALF)]],
                                  a0).wait()
            pltpu.async_copy(table_hbm.at[si.at[j + 1, pl.ds(0, HALF)]],
                             r0, g0)
            pltpu.make_async_copy(r1, acc.at[di.at[j, pl.ds(HALF, HALF)]],
                                  a1).wait()
            pltpu.async_copy(table_hbm.at[si.at[j + 1, pl.ds(HALF, HALF)]],
                             r1, g1)

        pltpu.make_async_copy(table_hbm.at[si.at[0, pl.ds(0, HALF)]],
                              r0, g0).wait()
        pltpu.make_async_copy(table_hbm.at[si.at[0, pl.ds(HALF, HALF)]],
                              r1, g1).wait()

        plsc.subcore_barrier()
        pltpu.sync_copy(acc.at[pl.ds(s * rpt, rpt)],
                        out_hbm.at[c].at[pl.ds(s * rpt, rpt)])

    return k(table, srcm, dstm, zeros_npd)


def _dinv_from(degp_ref, n):
    deg = degp_ref[0, :, 0] + degp_ref[1, :, 0] + 1.0
    return lax.rsqrt(deg)[:n, None]


def _tc1(x, W1, b1, degp):
    n = x.shape[0]
    d1 = W1.shape[1]

    def body(x_ref, w_ref, b_ref, degp_ref, h1s_ref):
        dinv = _dinv_from(degp_ref, n)
        h1 = jnp.dot(x_ref[...], w_ref[...],
                     preferred_element_type=jnp.float32) + b_ref[...]
        h1s_ref[...] = jnp.pad(h1 * dinv, ((0, 0), (0, DPAD - d1)))

    return pl.pallas_call(
        body,
        out_shape=jax.ShapeDtypeStruct((n, DPAD), jnp.float32),
    )(x, W1, b1.reshape(1, -1), degp)


def _tc2(acc1, degp, h1s, W2, b2):
    n = h1s.shape[0]
    d1 = W2.shape[0]
    d2 = W2.shape[1]

    def body(acc_ref, degp_ref, h1s_ref, w_ref, b_ref, out_ref):
        dinv = _dinv_from(degp_ref, n)
        agg = acc_ref[0, :n, :d1] + acc_ref[1, :n, :d1]
        l1 = jnp.maximum(dinv * agg + dinv * h1s_ref[:, :d1], 0.0)
        out2 = jnp.dot(l1, w_ref[...],
                       preferred_element_type=jnp.float32) + b_ref[...]
        out_ref[...] = jnp.pad(out2 * dinv, ((0, 0), (0, DPAD - d2)))

    return pl.pallas_call(
        body,
        out_shape=jax.ShapeDtypeStruct((n, DPAD), jnp.float32),
    )(acc1, degp, h1s, W2, b2.reshape(1, -1))


def _tc3(acc2, degp, out2s, d2):
    n = out2s.shape[0]

    def body(acc_ref, degp_ref, o2s_ref, out_ref):
        dinv = _dinv_from(degp_ref, n)
        agg = acc_ref[0, :n, :d2] + acc_ref[1, :n, :d2]
        out_ref[...] = dinv * agg + dinv * o2s_ref[:, :d2]

    return pl.pallas_call(
        body,
        out_shape=jax.ShapeDtypeStruct((n, d2), jnp.float32),
    )(acc2, degp, out2s)


def kernel(x, edge_index, W1, b1, W2, b2):
    n = x.shape[0]
    e = edge_index.shape[1]
    d2 = W2.shape[1]

    np_rows = _round_up(n + 1, NS * 8)
    nchunks = _round_up(e, LANES) // LANES
    cpt = _round_up(-(-nchunks // (NC * NS)), 8)
    e_pad = NC * NS * cpt * LANES

    src = edge_index[0].astype(jnp.int32)
    dst = edge_index[1].astype(jnp.int32)
    src = jnp.concatenate([src, jnp.zeros((e_pad - e,), jnp.int32)])
    dst = jnp.concatenate([dst, jnp.full((e_pad - e,), n, jnp.int32)])
    srcm = src.reshape(-1, LANES)
    dstm = dst.reshape(-1, LANES)

    ones_row = jnp.ones((LANES, DEGW), jnp.float32)
    zeros_deg = jnp.zeros((np_rows, DEGW), jnp.float32)
    zeros_d = jnp.zeros((np_rows, DPAD), jnp.float32)

    degp = _sc_degree(dstm, ones_row, zeros_deg, cpt=cpt, np_rows=np_rows)
    h1s = _tc1(x, W1, b1, degp)
    acc1 = _sc_propagate(h1s, srcm, dstm, zeros_d, cpt=cpt, np_rows=np_rows)
    out2s = _tc2(acc1, degp, h1s, W2, b2)
    acc2 = _sc_propagate(out2s, srcm, dstm, zeros_d, cpt=cpt, np_rows=np_rows)
    return _tc3(acc2, degp, out2s, d2)


# speedup vs baseline: 1.0468x; 1.0468x over previous
import functools

import jax
import jax.numpy as jnp
from jax import lax
from jax.experimental import pallas as pl
from jax.experimental.pallas import tpu as pltpu
from jax.experimental.pallas import tpu_sc as plsc

NC = 2
NS = 16
LANES = 128
HALF = 64
DPAD = 128
DEGW = 128

_vector_mesh = plsc.VectorSubcoreMesh(
    core_axis_name="core", subcore_axis_name="subcore")


def _round_up(a, b):
    return (a + b - 1) // b * b


def _sc_degree(dstm, ones_row, zeros_nd, *, cpt, np_rows):
    rpt = np_rows // NS

    @functools.partial(
        pl.kernel,
        out_type=jax.ShapeDtypeStruct((NC, np_rows, DEGW), jnp.float32),
        mesh=_vector_mesh,
        scratch_types=[
            pltpu.VMEM((cpt, LANES), jnp.int32),
            pltpu.VMEM((LANES, DEGW), jnp.float32),
            pltpu.VMEM_SHARED((np_rows, DEGW), jnp.float32),
        ],
    )
    def k(dstm_hbm, ones_hbm, zeros_hbm, out_hbm, di, ones_v, acc):
        c = lax.axis_index("core")
        s = lax.axis_index("subcore")
        wid = s * NC + c
        pltpu.sync_copy(dstm_hbm.at[pl.ds(wid * cpt, cpt)], di)
        pltpu.sync_copy(ones_hbm, ones_v)
        pltpu.sync_copy(zeros_hbm.at[pl.ds(s * rpt, rpt)],
                        acc.at[pl.ds(s * rpt, rpt)])
        plsc.subcore_barrier()

        @pl.loop(0, cpt)
        def _(j):
            pltpu.sync_copy(ones_v, acc.at[di.at[j]], add=True)

        plsc.subcore_barrier()
        pltpu.sync_copy(acc.at[pl.ds(s * rpt, rpt)],
                        out_hbm.at[c].at[pl.ds(s * rpt, rpt)])

    return k(dstm, ones_row, zeros_nd)


def _sc_propagate(table, srcm, dstm, zeros_npd, *, cpt, np_rows):
    rpt = np_rows // NS

    @functools.partial(
        pl.kernel,
        out_type=jax.ShapeDtypeStruct((NC, np_rows, DPAD), jnp.float32),
        mesh=_vector_mesh,
        scratch_types=[
            pltpu.VMEM((cpt + 1, LANES), jnp.int32),
            pltpu.VMEM((cpt, LANES), jnp.int32),
            pltpu.VMEM((HALF, DPAD), jnp.float32),
            pltpu.VMEM((HALF, DPAD), jnp.float32),
            pltpu.VMEM_SHARED((np_rows, DPAD), jnp.float32),
            pltpu.SemaphoreType.DMA,
            pltpu.SemaphoreType.DMA,
        ],
    )
    def k(table_hbm, srcm_hbm, dstm_hbm, zeros_hbm, out_hbm,
          si, di, r0, r1, acc, g0, g1):
        c = lax.axis_index("core")
        s = lax.axis_index("subcore")
        wid = s * NC + c
        pltpu.sync_copy(srcm_hbm.at[pl.ds(wid * cpt, cpt)],
                        si.at[pl.ds(0, cpt)])
        pltpu.sync_copy(srcm_hbm.at[pl.ds(wid * cpt, 1)],
                        si.at[pl.ds(cpt, 1)])
        pltpu.sync_copy(dstm_hbm.at[pl.ds(wid * cpt, cpt)], di)
        pltpu.sync_copy(zeros_hbm.at[pl.ds(s * rpt, rpt)],
                        acc.at[pl.ds(s * rpt, rpt)])
        plsc.subcore_barrier()

        pltpu.async_copy(table_hbm.at[si.at[0, pl.ds(0, HALF)]], r0, g0)
        pltpu.async_copy(table_hbm.at[si.at[0, pl.ds(HALF, HALF)]], r1, g1)

        @pl.loop(0, cpt)
        def _(j):
            pltpu.make_async_copy(table_hbm.at[si.at[j, pl.ds(0, HALF)]],
                                  r0, g0).wait()
            pltpu.sync_copy(r0, acc.at[di.at[j, pl.ds(0, HALF)]], add=True)
            pltpu.async_copy(table_hbm.at[si.at[j + 1, pl.ds(0, HALF)]],
                             r0, g0)

            pltpu.make_async_copy(table_hbm.at[si.at[j, pl.ds(HALF, HALF)]],
                                  r1, g1).wait()
            pltpu.sync_copy(r1, acc.at[di.at[j, pl.ds(HALF, HALF)]],
                            add=True)
            pltpu.async_copy(table_hbm.at[si.at[j + 1, pl.ds(HALF, HALF)]],
                             r1, g1)

        pltpu.make_async_copy(table_hbm.at[si.at[0, pl.ds(0, HALF)]],
                              r0, g0).wait()
        pltpu.make_async_copy(table_hbm.at[si.at[0, pl.ds(HALF, HALF)]],
                              r1, g1).wait()

        plsc.subcore_barrier()
        pltpu.sync_copy(acc.at[pl.ds(s * rpt, rpt)],
                        out_hbm.at[c].at[pl.ds(s * rpt, rpt)])

    return k(table, srcm, dstm, zeros_npd)


def _dinv_from(degp_ref, n):
    deg = degp_ref[0, :, 0] + degp_ref[1, :, 0] + 1.0
    return lax.rsqrt(deg)[:n, None]


def _tc1(x, W1, b1, degp):
    n = x.shape[0]
    d1 = W1.shape[1]

    def body(x_ref, w_ref, b_ref, degp_ref, h1s_ref):
        dinv = _dinv_from(degp_ref, n)
        h1 = jnp.dot(x_ref[...], w_ref[...],
                     preferred_element_type=jnp.float32) + b_ref[...]
        h1s_ref[...] = jnp.pad(h1 * dinv, ((0, 0), (0, DPAD - d1)))

    return pl.pallas_call(
        body,
        out_shape=jax.ShapeDtypeStruct((n, DPAD), jnp.float32),
    )(x, W1, b1.reshape(1, -1), degp)


def _tc2(acc1, degp, h1s, W2, b2):
    n = h1s.shape[0]
    d1 = W2.shape[0]
    d2 = W2.shape[1]

    def body(acc_ref, degp_ref, h1s_ref, w_ref, b_ref, out_ref):
        dinv = _dinv_from(degp_ref, n)
        agg = acc_ref[0, :n, :d1] + acc_ref[1, :n, :d1]
        l1 = jnp.maximum(dinv * agg + dinv * h1s_ref[:, :d1], 0.0)
        out2 = jnp.dot(l1, w_ref[...],
                       preferred_element_type=jnp.float32) + b_ref[...]
        out_ref[...] = jnp.pad(out2 * dinv, ((0, 0), (0, DPAD - d2)))

    return pl.pallas_call(
        body,
        out_shape=jax.ShapeDtypeStruct((n, DPAD), jnp.float32),
    )(acc1, degp, h1s, W2, b2.reshape(1, -1))


def _tc3(acc2, degp, out2s, d2):
    n = out2s.shape[0]

    def body(acc_ref, degp_ref, o2s_ref, out_ref):
        dinv = _dinv_from(degp_ref, n)
        agg = acc_ref[0, :n, :d2] + acc_ref[1, :n, :d2]
        out_ref[...] = dinv * agg + dinv * o2s_ref[:, :d2]

    return pl.pallas_call(
        body,
        out_shape=jax.ShapeDtypeStruct((n, d2), jnp.float32),
    )(acc2, degp, out2s)


def kernel(x, edge_index, W1, b1, W2, b2):
    n = x.shape[0]
    e = edge_index.shape[1]
    d2 = W2.shape[1]

    np_rows = _round_up(n + 1, NS * 8)
    nchunks = _round_up(e, LANES) // LANES
    cpt = _round_up(-(-nchunks // (NC * NS)), 8)
    e_pad = NC * NS * cpt * LANES

    src = edge_index[0].astype(jnp.int32)
    dst = edge_index[1].astype(jnp.int32)
    src = jnp.concatenate([src, jnp.zeros((e_pad - e,), jnp.int32)])
    dst = jnp.concatenate([dst, jnp.full((e_pad - e,), n, jnp.int32)])
    srcm = src.reshape(-1, LANES)
    dstm = dst.reshape(-1, LANES)

    ones_row = jnp.ones((LANES, DEGW), jnp.float32)
    zeros_deg = jnp.zeros((np_rows, DEGW), jnp.float32)
    zeros_d = jnp.zeros((np_rows, DPAD), jnp.float32)

    degp = _sc_degree(dstm, ones_row, zeros_deg, cpt=cpt, np_rows=np_rows)
    h1s = _tc1(x, W1, b1, degp)
    acc1 = _sc_propagate(h1s, srcm, dstm, zeros_d, cpt=cpt, np_rows=np_rows)
    out2s = _tc2(acc1, degp, h1s, W2, b2)
    acc2 = _sc_propagate(out2s, srcm, dstm, zeros_d, cpt=cpt, np_rows=np_rows)
    return _tc3(acc2, degp, out2s, d2)
